# baseline (speedup 1.0000x reference)
import functools

import jax
import jax.numpy as jnp
from jax import lax
from jax.experimental import pallas as pl
from jax.experimental.pallas import tpu as pltpu
from jax.experimental.pallas import tpu_sc as plsc

B, F, D = 4096, 26, 32
NC, NS = 2, 16
NW = NC * NS
RPW = B // NW
NL = RPW // 16
NCHUNK = 4
DC = D // NCHUNK
BH = B // 2
TCBLK = 256


def _tree_sum(vals):
    vals = list(vals)
    while len(vals) > 1:
        nxt = [vals[i] + vals[i + 1] for i in range(0, len(vals) - 1, 2)]
        if len(vals) % 2:
            nxt.append(vals[-1])
        vals = nxt
    return vals[0]


def _fm_sc_body(st_hbm, et_hbm, w_hbm, fo_hbm, so_hbm,
                slab_v, gath_v, emb_v, out_v, so_v,
                gsem, s0, s1, s2, s3):
    wid = lax.axis_index("s") * NC + lax.axis_index("c")
    base = wid * RPW
    with jax.named_scope("slab_copy"):
        pltpu.sync_copy(st_hbm.at[:, pl.ds(base, RPW)], slab_v)

    dense = wid < NW // 2
    dbase = jnp.where(dense, wid * RPW, 0)

    sems = (s0, s1, s2, s3)

    @pl.when(dense)
    def _():
        for c in range(NCHUNK):
            pltpu.async_copy(
                et_hbm.at[:, pl.ds(c * DC, DC), pl.ds(dbase, RPW)],
                emb_v.at[:, pl.ds(c * DC, DC), :], sems[c])

    with jax.named_scope("gather_fire"):
        gathers = [
            pltpu.async_copy(w_hbm.at[0].at[slab_v.at[f]], gath_v.at[f], gsem)
            for f in range(F)
        ]

    zero16 = jnp.zeros((16,), jnp.float32)

    @pl.when(dense)
    def _():
        def dbody(dd, accs):
            for c in range(NCHUNK):
                @pl.when(dd == c * DC)
                def _(c=c):
                    pltpu.make_async_copy(
                        et_hbm.at[:, pl.ds(c * DC, DC), pl.ds(dbase, RPW)],
                        emb_v.at[:, pl.ds(c * DC, DC), :], sems[c]).wait()
            new = []
            for l in range(NL):
                vs = [emb_v[f, dd, pl.ds(l * 16, 16)] for f in range(F)]
                s = _tree_sum(vs)
                q = _tree_sum([v * v for v in vs])
                new.append(accs[l] + (s * s - q))
            return tuple(new)

        with jax.named_scope("dense_loop"):
            accs = lax.fori_loop(0, D, dbody, (zero16,) * NL)
        for l in range(NL):
            so_v[pl.ds(l * 16, 16)] = 0.5 * accs[l]
        pltpu.sync_copy(so_v, so_hbm.at[pl.ds(dbase, RPW)])

    with jax.named_scope("gather_drain"):
        for g in gathers:
            g.wait()

    with jax.named_scope("fo_sum"):
        for l in range(NL):
            fo = zero16
            for f in range(F):
                fo = fo + gath_v[f, pl.ds(l * 16, 16)]
            out_v[pl.ds(l * 16, 16)] = fo

    pltpu.sync_copy(out_v, fo_hbm.at[pl.ds(base, RPW)])


def _fm_tc_body(et_ref, so_ref):
    x = et_ref[...]
    s = jnp.sum(x, axis=0)
    q = jnp.sum(x * x, axis=0)
    so_ref[...] = 0.5 * jnp.sum(s * s - q, axis=0)


@jax.jit
def kernel(sparse_inputs, embed_inputs, w):
    run_sc = pl.kernel(
        _fm_sc_body,
        out_type=(jax.ShapeDtypeStruct((B,), jnp.float32),
                  jax.ShapeDtypeStruct((BH,), jnp.float32)),
        mesh=plsc.VectorSubcoreMesh(core_axis_name="c", subcore_axis_name="s"),
        scratch_types=[
            pltpu.VMEM((F, RPW), jnp.int32),
            pltpu.VMEM((F, RPW), jnp.float32),
            pltpu.VMEM((F, D, RPW), jnp.float32),
            pltpu.VMEM((RPW,), jnp.float32),
            pltpu.VMEM((RPW,), jnp.float32),
            pltpu.SemaphoreType.DMA,
            pltpu.SemaphoreType.DMA,
            pltpu.SemaphoreType.DMA,
            pltpu.SemaphoreType.DMA,
            pltpu.SemaphoreType.DMA,
        ],
        compiler_params=pltpu.CompilerParams(needs_layout_passes=False),
    )
    st = sparse_inputs.T
    et = embed_inputs.transpose(1, 2, 0)
    wt = w.T

    fo, so_lo = run_sc(st, et, wt)

    nblk = BH // TCBLK
    so_hi = pl.pallas_call(
        _fm_tc_body,
        grid=(nblk,),
        in_specs=[pl.BlockSpec((F, D, TCBLK),
                               lambda i: (0, 0, i + nblk))],
        out_specs=pl.BlockSpec((TCBLK,), lambda i: (i,)),
        out_shape=jax.ShapeDtypeStruct((BH,), jnp.float32),
    )(et)

    so = jnp.concatenate([so_lo, so_hi])
    return (fo + so).reshape(B, 1)


# speedup vs baseline: 1.0921x; 1.0921x over previous
import functools

import jax
import jax.numpy as jnp
from jax import lax
from jax.experimental import pallas as pl
from jax.experimental.pallas import tpu as pltpu
from jax.experimental.pallas import tpu_sc as plsc

B, F, D = 4096, 26, 32
NC, NS = 2, 16
NW = NC * NS
RPW = B // NW
NL = RPW // 16
DH = D // 2
NCHUNK = 2
DC = DH // NCHUNK
BH = B // 2
TCBLK = 256


def _tree_sum(vals):
    vals = list(vals)
    while len(vals) > 1:
        nxt = [vals[i] + vals[i + 1] for i in range(0, len(vals) - 1, 2)]
        if len(vals) % 2:
            nxt.append(vals[-1])
        vals = nxt
    return vals[0]


def _fm_sc_body(st_hbm, et_hbm, w_hbm, fo_hbm, so_hbm,
                slab_v, gath_v, emb_v, out_v, so_v, gsem, s0, s1):
    wid = lax.axis_index("s") * NC + lax.axis_index("c")
    base = wid * RPW
    dhalf = wid // (NW // 2)
    blk = wid % (NW // 2)
    dbase = blk * RPW
    d0 = dhalf * DH

    sems = (s0, s1)
    chunk_views = [
        (et_hbm.at[:, pl.ds(d0 + c * DC, DC), pl.ds(dbase, RPW)],
         emb_v.at[:, pl.ds(c * DC, DC), :])
        for c in range(NCHUNK)
    ]
    for c in range(NCHUNK):
        pltpu.async_copy(chunk_views[c][0], chunk_views[c][1], sems[c])

    pltpu.sync_copy(st_hbm.at[:, pl.ds(base, RPW)], slab_v)

    gathers = [
        pltpu.async_copy(w_hbm.at[0].at[slab_v.at[f]], gath_v.at[f], gsem)
        for f in range(F)
    ]

    zero16 = jnp.zeros((16,), jnp.float32)

    def dbody(dd, accs):
        for c in range(NCHUNK):
            @pl.when(dd == c * DC)
            def _(c=c):
                pltpu.make_async_copy(chunk_views[c][0], chunk_views[c][1],
                                      sems[c]).wait()
        new = []
        for l in range(NL):
            vs = [emb_v[f, dd, pl.ds(l * 16, 16)] for f in range(F)]
            s = _tree_sum(vs)
            q = _tree_sum([v * v for v in vs])
            new.append(accs[l] + (s * s - q))
        return tuple(new)

    accs = lax.fori_loop(0, DH, dbody, (zero16,) * NL)
    for l in range(NL):
        so_v[pl.ds(l * 16, 16)] = 0.5 * accs[l]
    pltpu.sync_copy(so_v, so_hbm.at[pl.ds(dhalf * BH + dbase, RPW)])

    for g in gathers:
        g.wait()

    for l in range(NL):
        fo = _tree_sum([gath_v[f, pl.ds(l * 16, 16)] for f in range(F)])
        out_v[pl.ds(l * 16, 16)] = fo

    pltpu.sync_copy(out_v, fo_hbm.at[pl.ds(base, RPW)])


def _fm_tc_body(et_ref, so_ref):
    x = et_ref[...]
    s = jnp.sum(x, axis=0)
    q = jnp.sum(x * x, axis=0)
    so_ref[...] = 0.5 * jnp.sum(s * s - q, axis=0)


@jax.jit
def kernel(sparse_inputs, embed_inputs, w):
    run_sc = pl.kernel(
        _fm_sc_body,
        out_type=(jax.ShapeDtypeStruct((B,), jnp.float32),
                  jax.ShapeDtypeStruct((2 * BH,), jnp.float32)),
        mesh=plsc.VectorSubcoreMesh(core_axis_name="c", subcore_axis_name="s"),
        scratch_types=[
            pltpu.VMEM((F, RPW), jnp.int32),
            pltpu.VMEM((F, RPW), jnp.float32),
            pltpu.VMEM((F, DH, RPW), jnp.float32),
            pltpu.VMEM((RPW,), jnp.float32),
            pltpu.VMEM((RPW,), jnp.float32),
            pltpu.SemaphoreType.DMA,
            pltpu.SemaphoreType.DMA,
            pltpu.SemaphoreType.DMA,
        ],
        compiler_params=pltpu.CompilerParams(needs_layout_passes=False),
    )
    st = sparse_inputs.T
    et = embed_inputs.transpose(1, 2, 0)
    wt = w.T

    fo, so_parts = run_sc(st, et, wt)

    nblk = BH // TCBLK
    so_hi = pl.pallas_call(
        _fm_tc_body,
        grid=(nblk,),
        in_specs=[pl.BlockSpec((F, D, TCBLK),
                               lambda i: (0, 0, i + nblk))],
        out_specs=pl.BlockSpec((TCBLK,), lambda i: (i,)),
        out_shape=jax.ShapeDtypeStruct((BH,), jnp.float32),
    )(et)

    so_lo = so_parts[:BH] + so_parts[BH:]
    so = jnp.concatenate([so_lo, so_hi])
    return (fo + so).reshape(B, 1)
